# baseline (speedup 1.0000x reference)
import functools

import jax
import jax.numpy as jnp
from jax import lax
from jax.experimental import pallas as pl
from jax.experimental.pallas import tpu as pltpu
from jax.experimental.pallas import tpu_sc as plsc

N_NODES = 10000
N_EDGES = 160000
D_IN = 256
D_HID = 256
N_CLASSES = 40
N_EXPERTS = 8
D_P = N_EXPERTS * N_CLASSES

NC = 2
NS = 16
E_PER_TILE = 5120
E_PAD = NC * NS * E_PER_TILE
ROWS_PER_TILE = 632
N_ACC = ROWS_PER_TILE * NS
TRASH = N_NODES + 4

BLK = 1000
GRID = N_NODES // BLK


def _make_segsum(d, chunk, super_, nsuper):
    mesh = plsc.VectorSubcoreMesh(core_axis_name="c", subcore_axis_name="s")
    pairs = nsuper // 2

    @functools.partial(
        pl.kernel,
        mesh=mesh,
        out_type=jax.ShapeDtypeStruct((NC, N_ACC, d), jnp.bfloat16),
        compiler_params=pltpu.CompilerParams(use_tc_tiling_on_sc=False),
        scratch_types=[
            pltpu.VMEM((super_, 2, chunk), jnp.int32),
            pltpu.VMEM((super_, 2, chunk), jnp.int32),
            pltpu.VMEM((chunk, d), jnp.bfloat16),
            pltpu.VMEM((chunk, d), jnp.bfloat16),
            pltpu.VMEM_SHARED((N_ACC, d), jnp.bfloat16),
            pltpu.SemaphoreType.DMA,
            pltpu.SemaphoreType.DMA,
            pltpu.SemaphoreType.DMA,
            pltpu.SemaphoreType.DMA,
        ],
    )
    def seg_kernel(table, idx_hbm, zrows, out_hbm,
                   idx0, idx1, rows0, rows1, acc,
                   semi0, semi1, semr0, semr1):
        c = lax.axis_index("c")
        s = lax.axis_index("s")
        ih = idx_hbm.at[c, s]
        idx_bufs = (idx0, idx1)
        idx_sems = (semi0, semi1)
        row_bufs = (rows0, rows1)
        row_sems = (semr0, semr1)

        pltpu.async_copy(ih.at[0], idx0, semi0)
        pltpu.async_copy(ih.at[1], idx1, semi1)
        pltpu.sync_copy(zrows, acc.at[pl.ds(s * ROWS_PER_TILE, ROWS_PER_TILE)])
        plsc.subcore_barrier()

        def run_super(j, b, not_last):
            idx = idx_bufs[b]
            cps = [None, None]
            cps[0] = pltpu.async_copy(table.at[idx.at[0, 0]], rows0, semr0)
            for ci in range(super_):
                rb = ci % 2
                if ci + 1 < super_:
                    nb = (ci + 1) % 2
                    cps[nb] = pltpu.async_copy(
                        table.at[idx.at[ci + 1, 0]], row_bufs[nb], row_sems[nb])
                cps[rb].wait()
                pltpu.sync_copy(row_bufs[rb], acc.at[idx.at[ci, 1]], add=True)
            @pl.when(not_last)
            def _():
                pltpu.async_copy(ih.at[j + 2], idx_bufs[b], idx_sems[b])

        def pair(k, carry):
            not_last = k < pairs - 1
            pltpu.make_async_copy(ih.at[2 * k], idx0, semi0).wait()
            run_super(2 * k, 0, not_last)
            pltpu.make_async_copy(ih.at[2 * k + 1], idx1, semi1).wait()
            run_super(2 * k + 1, 1, not_last)
            return carry

        lax.fori_loop(0, pairs, pair, 0)
        plsc.subcore_barrier()
        pltpu.sync_copy(acc.at[pl.ds(s * ROWS_PER_TILE, ROWS_PER_TILE)],
                        out_hbm.at[c, pl.ds(s * ROWS_PER_TILE, ROWS_PER_TILE)])

    return seg_kernel


CFG = (80, 8, 8)
_segsum_x = _make_segsum(D_IN, *CFG)
_segsum_p = _make_segsum(D_P, *CFG)


def _front_body(x_ref, agg1_ref, noise_ref, wg_ref, wn_ref, thr_ref,
                w1_ref, w2_ref, gates_ref, p_ref):
    x = x_ref[...]
    agg1 = (agg1_ref[0] + agg1_ref[1]).astype(jnp.float32)
    z = x + agg1
    clean = jnp.dot(x, wg_ref[...], preferred_element_type=jnp.float32)
    nlog = jnp.dot(x, wn_ref[...], preferred_element_type=jnp.float32)
    std = jax.nn.softplus(nlog) + 1e-2
    noisy = clean + noise_ref[...] * std
    scores = noisy - thr_ref[...]
    open_mask = (scores > 0).astype(jnp.float32)
    m = jnp.max(noisy, axis=1, keepdims=True)
    ex = jnp.exp(noisy - m)
    sm = ex / jnp.sum(ex, axis=1, keepdims=True)
    raw = sm * open_mask
    gates_ref[...] = raw / (jnp.sum(raw, axis=1, keepdims=True) + 1e-9)
    zb = z.astype(jnp.bfloat16)
    for e in range(N_EXPERTS):
        h = jnp.maximum(
            jnp.dot(zb, w1_ref[e].astype(jnp.bfloat16),
                    preferred_element_type=jnp.float32), 0.0)
        p = jnp.dot(h.astype(jnp.bfloat16), w2_ref[e].astype(jnp.bfloat16),
                    preferred_element_type=jnp.float32)
        p_ref[:, e * N_CLASSES:(e + 1) * N_CLASSES] = p.astype(jnp.bfloat16)


def _combine_body(gates_ref, p_ref, agg2_ref, y_ref):
    g = gates_ref[...]
    p = p_ref[...].astype(jnp.float32)
    agg2 = (agg2_ref[0] + agg2_ref[1]).astype(jnp.float32)
    o = agg2 + p
    acc = jnp.zeros((BLK, N_CLASSES), jnp.float32)
    for e in range(N_EXPERTS):
        acc = acc + g[:, e:e + 1] * o[:, e * N_CLASSES:(e + 1) * N_CLASSES]
    y_ref[...] = acc


def kernel(x, edge_index, noise, w_gate, w_noise, gate_threshold, W1, W2):
    src = edge_index[0].astype(jnp.int32)
    dst = edge_index[1].astype(jnp.int32)
    pad = E_PAD - N_EDGES
    src_p = jnp.concatenate([src, jnp.zeros((pad,), jnp.int32)])
    dst_p = jnp.concatenate([dst, jnp.full((pad,), TRASH, jnp.int32)])
    def build_idx(chunk, super_, nsuper):
        srcE = src_p.reshape(NS, nsuper, super_, NC, chunk)
        dstE = dst_p.reshape(NS, nsuper, super_, NC, chunk)
        coff = (jnp.arange(NC, dtype=jnp.int32) * N_NODES)[None, None, None, :, None]
        idx6 = jnp.stack([srcE + coff, dstE])
        return idx6.transpose(4, 1, 2, 3, 0, 5)

    idx = build_idx(*CFG)

    ztab_x = jnp.zeros((ROWS_PER_TILE, D_IN), jnp.bfloat16)
    ztab_p = jnp.zeros((ROWS_PER_TILE, D_P), jnp.bfloat16)

    xb = x.astype(jnp.bfloat16)
    x_tab = jnp.concatenate([xb, xb])
    agg1_s = _segsum_x(x_tab, idx, ztab_x)

    thr = gate_threshold.reshape(1, N_EXPERTS)
    gates, p_out = pl.pallas_call(
        _front_body,
        grid=(GRID,),
        in_specs=[
            pl.BlockSpec((BLK, D_IN), lambda i: (i, 0)),
            pl.BlockSpec((NC, BLK, D_IN), lambda i: (0, i, 0)),
            pl.BlockSpec((BLK, N_EXPERTS), lambda i: (i, 0)),
            pl.BlockSpec((D_IN, N_EXPERTS), lambda i: (0, 0)),
            pl.BlockSpec((D_IN, N_EXPERTS), lambda i: (0, 0)),
            pl.BlockSpec((1, N_EXPERTS), lambda i: (0, 0)),
            pl.BlockSpec((N_EXPERTS, D_IN, D_HID), lambda i: (0, 0, 0)),
            pl.BlockSpec((N_EXPERTS, D_HID, N_CLASSES), lambda i: (0, 0, 0)),
        ],
        out_specs=[
            pl.BlockSpec((BLK, N_EXPERTS), lambda i: (i, 0)),
            pl.BlockSpec((BLK, D_P), lambda i: (i, 0)),
        ],
        out_shape=[
            jax.ShapeDtypeStruct((N_NODES, N_EXPERTS), jnp.float32),
            jax.ShapeDtypeStruct((N_NODES, D_P), jnp.bfloat16),
        ],
    )(x, agg1_s, noise, w_gate, w_noise, thr, W1, W2)

    p_tab = jnp.concatenate([p_out, p_out])
    agg2_s = _segsum_p(p_tab, idx, ztab_p)

    y = pl.pallas_call(
        _combine_body,
        grid=(GRID,),
        in_specs=[
            pl.BlockSpec((BLK, N_EXPERTS), lambda i: (i, 0)),
            pl.BlockSpec((BLK, D_P), lambda i: (i, 0)),
            pl.BlockSpec((NC, BLK, D_P), lambda i: (0, i, 0)),
        ],
        out_specs=pl.BlockSpec((BLK, N_CLASSES), lambda i: (i, 0)),
        out_shape=jax.ShapeDtypeStruct((N_NODES, N_CLASSES), jnp.float32),
    )(gates, p_out, agg2_s)
    return y


# speedup vs baseline: 1.0167x; 1.0167x over previous
import functools

import jax
import jax.numpy as jnp
from jax import lax
from jax.experimental import pallas as pl
from jax.experimental.pallas import tpu as pltpu
from jax.experimental.pallas import tpu_sc as plsc

N_NODES = 10000
N_EDGES = 160000
D_IN = 256
D_HID = 256
N_CLASSES = 40
N_EXPERTS = 8
D_P = N_EXPERTS * N_CLASSES

NC = 2
NS = 16
E_PER_TILE = 5120
E_PAD = NC * NS * E_PER_TILE
ROWS_PER_TILE = 632
N_ACC = ROWS_PER_TILE * NS
TRASH = N_NODES + 4

BLK = 2000
GRID = N_NODES // BLK


def _make_segsum(d, chunk, super_, nsuper):
    mesh = plsc.VectorSubcoreMesh(core_axis_name="c", subcore_axis_name="s")
    pairs = nsuper // 2

    @functools.partial(
        pl.kernel,
        mesh=mesh,
        out_type=jax.ShapeDtypeStruct((NC, N_ACC, d), jnp.bfloat16),
        compiler_params=pltpu.CompilerParams(use_tc_tiling_on_sc=False),
        scratch_types=[
            pltpu.VMEM((super_, 2, chunk), jnp.int32),
            pltpu.VMEM((super_, 2, chunk), jnp.int32),
            pltpu.VMEM((chunk, d), jnp.bfloat16),
            pltpu.VMEM((chunk, d), jnp.bfloat16),
            pltpu.VMEM_SHARED((N_ACC, d), jnp.bfloat16),
            pltpu.SemaphoreType.DMA,
            pltpu.SemaphoreType.DMA,
            pltpu.SemaphoreType.DMA,
            pltpu.SemaphoreType.DMA,
        ],
    )
    def seg_kernel(table, idx_hbm, zrows, out_hbm,
                   idx0, idx1, rows0, rows1, acc,
                   semi0, semi1, semr0, semr1):
        c = lax.axis_index("c")
        s = lax.axis_index("s")
        ih = idx_hbm.at[c, s]
        idx_bufs = (idx0, idx1)
        idx_sems = (semi0, semi1)
        row_bufs = (rows0, rows1)
        row_sems = (semr0, semr1)

        pltpu.async_copy(ih.at[0], idx0, semi0)
        pltpu.async_copy(ih.at[1], idx1, semi1)
        pltpu.sync_copy(zrows, acc.at[pl.ds(s * ROWS_PER_TILE, ROWS_PER_TILE)])
        plsc.subcore_barrier()

        def run_super(j, b, not_last):
            idx = idx_bufs[b]
            cps = [None, None]
            cps[0] = pltpu.async_copy(table.at[idx.at[0, 0]], rows0, semr0)
            for ci in range(super_):
                rb = ci % 2
                if ci + 1 < super_:
                    nb = (ci + 1) % 2
                    cps[nb] = pltpu.async_copy(
                        table.at[idx.at[ci + 1, 0]], row_bufs[nb], row_sems[nb])
                cps[rb].wait()
                pltpu.sync_copy(row_bufs[rb], acc.at[idx.at[ci, 1]], add=True)
            @pl.when(not_last)
            def _():
                pltpu.async_copy(ih.at[j + 2], idx_bufs[b], idx_sems[b])

        def pair(k, carry):
            not_last = k < pairs - 1
            pltpu.make_async_copy(ih.at[2 * k], idx0, semi0).wait()
            run_super(2 * k, 0, not_last)
            pltpu.make_async_copy(ih.at[2 * k + 1], idx1, semi1).wait()
            run_super(2 * k + 1, 1, not_last)
            return carry

        lax.fori_loop(0, pairs, pair, 0)
        plsc.subcore_barrier()
        pltpu.sync_copy(acc.at[pl.ds(s * ROWS_PER_TILE, ROWS_PER_TILE)],
                        out_hbm.at[c, pl.ds(s * ROWS_PER_TILE, ROWS_PER_TILE)])

    return seg_kernel


CFG_X = (128, 4, 10)
CFG_P = (80, 8, 8)
_segsum_x = _make_segsum(D_IN, *CFG_X)
_segsum_p = _make_segsum(D_P, *CFG_P)


def _front_body(x_ref, agg1_ref, noise_ref, wg_ref, wn_ref, thr_ref,
                w1_ref, w2_ref, gates_ref, p_ref):
    x = x_ref[...]
    agg1 = (agg1_ref[0] + agg1_ref[1]).astype(jnp.float32)
    z = x + agg1
    clean = jnp.dot(x, wg_ref[...], preferred_element_type=jnp.float32)
    nlog = jnp.dot(x, wn_ref[...], preferred_element_type=jnp.float32)
    std = jax.nn.softplus(nlog) + 1e-2
    noisy = clean + noise_ref[...] * std
    scores = noisy - thr_ref[...]
    open_mask = (scores > 0).astype(jnp.float32)
    m = jnp.max(noisy, axis=1, keepdims=True)
    ex = jnp.exp(noisy - m)
    sm = ex / jnp.sum(ex, axis=1, keepdims=True)
    raw = sm * open_mask
    gates_ref[...] = raw / (jnp.sum(raw, axis=1, keepdims=True) + 1e-9)
    zb = z.astype(jnp.bfloat16)
    for e in range(N_EXPERTS):
        h = jnp.maximum(
            jnp.dot(zb, w1_ref[e].astype(jnp.bfloat16),
                    preferred_element_type=jnp.float32), 0.0)
        p = jnp.dot(h.astype(jnp.bfloat16), w2_ref[e].astype(jnp.bfloat16),
                    preferred_element_type=jnp.float32)
        p_ref[:, e * N_CLASSES:(e + 1) * N_CLASSES] = p.astype(jnp.bfloat16)


def _combine_body(gates_ref, p_ref, agg2_ref, y_ref):
    g = gates_ref[...]
    p = p_ref[...].astype(jnp.float32)
    agg2 = (agg2_ref[0] + agg2_ref[1]).astype(jnp.float32)
    o = agg2 + p
    acc = jnp.zeros((BLK, N_CLASSES), jnp.float32)
    for e in range(N_EXPERTS):
        acc = acc + g[:, e:e + 1] * o[:, e * N_CLASSES:(e + 1) * N_CLASSES]
    y_ref[...] = acc


def kernel(x, edge_index, noise, w_gate, w_noise, gate_threshold, W1, W2):
    src = edge_index[0].astype(jnp.int32)
    dst = edge_index[1].astype(jnp.int32)
    pad = E_PAD - N_EDGES
    src_p = jnp.concatenate([src, jnp.zeros((pad,), jnp.int32)])
    dst_p = jnp.concatenate([dst, jnp.full((pad,), TRASH, jnp.int32)])
    def build_idx(chunk, super_, nsuper):
        srcE = src_p.reshape(NS, nsuper, super_, NC, chunk)
        dstE = dst_p.reshape(NS, nsuper, super_, NC, chunk)
        coff = (jnp.arange(NC, dtype=jnp.int32) * N_NODES)[None, None, None, :, None]
        idx6 = jnp.stack([srcE + coff, dstE])
        return idx6.transpose(4, 1, 2, 3, 0, 5)

    idx_x = build_idx(*CFG_X)
    idx_p = build_idx(*CFG_P)

    ztab_x = jnp.zeros((ROWS_PER_TILE, D_IN), jnp.bfloat16)
    ztab_p = jnp.zeros((ROWS_PER_TILE, D_P), jnp.bfloat16)

    xb = x.astype(jnp.bfloat16)
    x_tab = jnp.concatenate([xb, xb])
    agg1_s = _segsum_x(x_tab, idx_x, ztab_x)

    thr = gate_threshold.reshape(1, N_EXPERTS)
    gates, p_out = pl.pallas_call(
        _front_body,
        grid=(GRID,),
        in_specs=[
            pl.BlockSpec((BLK, D_IN), lambda i: (i, 0)),
            pl.BlockSpec((NC, BLK, D_IN), lambda i: (0, i, 0)),
            pl.BlockSpec((BLK, N_EXPERTS), lambda i: (i, 0)),
            pl.BlockSpec((D_IN, N_EXPERTS), lambda i: (0, 0)),
            pl.BlockSpec((D_IN, N_EXPERTS), lambda i: (0, 0)),
            pl.BlockSpec((1, N_EXPERTS), lambda i: (0, 0)),
            pl.BlockSpec((N_EXPERTS, D_IN, D_HID), lambda i: (0, 0, 0)),
            pl.BlockSpec((N_EXPERTS, D_HID, N_CLASSES), lambda i: (0, 0, 0)),
        ],
        out_specs=[
            pl.BlockSpec((BLK, N_EXPERTS), lambda i: (i, 0)),
            pl.BlockSpec((BLK, D_P), lambda i: (i, 0)),
        ],
        out_shape=[
            jax.ShapeDtypeStruct((N_NODES, N_EXPERTS), jnp.float32),
            jax.ShapeDtypeStruct((N_NODES, D_P), jnp.bfloat16),
        ],
    )(x, agg1_s, noise, w_gate, w_noise, thr, W1, W2)

    p_tab = jnp.concatenate([p_out, p_out])
    agg2_s = _segsum_p(p_tab, idx_p, ztab_p)

    y = pl.pallas_call(
        _combine_body,
        grid=(GRID,),
        in_specs=[
            pl.BlockSpec((BLK, N_EXPERTS), lambda i: (i, 0)),
            pl.BlockSpec((BLK, D_P), lambda i: (i, 0)),
            pl.BlockSpec((NC, BLK, D_P), lambda i: (0, i, 0)),
        ],
        out_specs=pl.BlockSpec((BLK, N_CLASSES), lambda i: (i, 0)),
        out_shape=jax.ShapeDtypeStruct((N_NODES, N_CLASSES), jnp.float32),
    )(gates, p_out, agg2_s)
    return y
